# baseline (speedup 1.0000x reference)
import functools

import jax
import jax.numpy as jnp
from jax import lax
from jax.experimental import pallas as pl
from jax.experimental.pallas import tpu as pltpu
from jax.experimental.pallas import tpu_sc as plsc

N = 10000
E = 320000
D = 128
RATE = 0.5
ALPHA = 0.1

NS = 16
CH = 80
EPT = E // NS
NCHUNK = EPT // CH
RPT = 624
TAIL = N - NS * RPT

_MESH = plsc.VectorSubcoreMesh(
    core_axis_name="c", subcore_axis_name="s", num_cores=2, num_subcores=16)


def _leaky(x):
    return jnp.where(x > 0, x, ALPHA * x)


DH = D // 2


NBUF = 5
PREF = 4
NBLK = NCHUNK // NBUF


def _spmm_domain(sup_a, sup_b, src_hbm, dst_hbm, val16_hbm, zeros_hbm,
                 out_a, out_b, idx_src, idx_dst, bufs, vbufs, gs, ss, accum):
    sid = lax.axis_index("s")
    pltpu.sync_copy(src_hbm.at[sid], idx_src)
    pltpu.sync_copy(dst_hbm.at[sid], idx_dst)

    for sup_hbm, out_hbm in ((sup_a, out_a), (sup_b, out_b)):

        def gstart(j, b):
            pltpu.make_async_copy(sup_hbm.at[idx_src.at[j]], bufs[b],
                                  gs[b]).start()
            pltpu.make_async_copy(val16_hbm.at[sid, j], vbufs[b],
                                  gs[b]).start()

        def gwait(j, b):
            pltpu.make_async_copy(sup_hbm.at[idx_src.at[j]], bufs[b],
                                  gs[b]).wait()
            pltpu.make_async_copy(val16_hbm.at[sid, j], vbufs[b],
                                  gs[b]).wait()

        def sstart(j, b):
            pltpu.make_async_copy(bufs[b], accum.at[idx_dst.at[j]],
                                  ss[b]).start(add=True)

        def swait(j, b):
            pltpu.make_async_copy(bufs[b], accum.at[idx_dst.at[j]],
                                  ss[b]).wait()

        def mul(j, b):
            rows = bufs[b]
            vb = vbufs[b]

            def ebody(i, _):
                for u in range(2):
                    e = i * 2 + u
                    v16 = vb[e, :]
                    for k in range(DH // 16):
                        sl = pl.ds(k * 16, 16)
                        rows[e, sl] = rows[e, sl] * v16
                return 0

            lax.fori_loop(0, CH // 2, ebody, 0)

        pltpu.sync_copy(zeros_hbm.at[pl.ds(sid * RPT, RPT)],
                        accum.at[pl.ds(sid * RPT, RPT)])

        @pl.when(sid == NS - 1)
        def _():
            pltpu.sync_copy(zeros_hbm.at[pl.ds(NS * RPT, TAIL)],
                            accum.at[pl.ds(NS * RPT, TAIL)])

        plsc.subcore_barrier()

        for b in range(PREF):
            gstart(b, b)

        def block(g, _):
            j0 = g * NBUF
            for b in range(NBUF):
                j = j0 + b
                gwait(j, b)
                mul(j, b)
                sstart(j, b)
                bn = (b + PREF) % NBUF
                jn = j + PREF

                @pl.when(jn < NCHUNK)
                def _():
                    @pl.when(j >= NBUF - PREF)
                    def _():
                        swait(j - (NBUF - PREF), bn)

                    gstart(jn, bn)
            return 0

        lax.fori_loop(0, NBLK, block, 0)

        for b in range(NBUF):
            swait(NCHUNK - NBUF + b, b)

        plsc.subcore_barrier()
        pltpu.sync_copy(accum.at[pl.ds(sid * RPT, RPT)],
                        out_hbm.at[pl.ds(sid * RPT, RPT)])

        @pl.when(sid == NS - 1)
        def _():
            pltpu.sync_copy(accum.at[pl.ds(NS * RPT, TAIL)],
                            out_hbm.at[pl.ds(NS * RPT, TAIL)])

        plsc.subcore_barrier()


@functools.partial(
    pl.kernel,
    out_type=tuple(jax.ShapeDtypeStruct((N, DH), jnp.float32)
                   for _ in range(4)),
    mesh=_MESH,
    scratch_types=[
        pltpu.VMEM((NCHUNK, CH), jnp.int32),
        pltpu.VMEM((NCHUNK, CH), jnp.int32),
        [pltpu.VMEM((CH, DH), jnp.float32)] * NBUF,
        [pltpu.VMEM((CH, 16), jnp.float32)] * NBUF,
        [pltpu.SemaphoreType.DMA] * NBUF,
        [pltpu.SemaphoreType.DMA] * NBUF,
        pltpu.VMEM_SHARED((N, DH), jnp.float32),
    ],
    compiler_params=pltpu.CompilerParams(use_tc_tiling_on_sc=False),
)
def _spmm_kernel(sup_s_a, sup_s_b, src_s, dst_s, val_s,
                 sup_t_a, sup_t_b, src_t, dst_t, val_t,
                 zeros_hbm, out_s_a, out_s_b, out_t_a, out_t_b,
                 idx_src, idx_dst, bufs, vbufs, gs, ss, accum):
    c = lax.axis_index("c")

    @pl.when(c == 0)
    def _():
        _spmm_domain(sup_s_a, sup_s_b, src_s, dst_s, val_s, zeros_hbm,
                     out_s_a, out_s_b, idx_src, idx_dst, bufs, vbufs, gs, ss,
                     accum)

    @pl.when(c == 1)
    def _():
        _spmm_domain(sup_t_a, sup_t_b, src_t, dst_t, val_t, zeros_hbm,
                     out_t_a, out_t_b, idx_src, idx_dst, bufs, vbufs, gs, ss,
                     accum)


def _spmm_pair(sup_s, src_s, dst_s, val_s, sup_t, src_t, dst_t, val_t, zeros):
    o = _spmm_kernel(sup_s[:, :DH], sup_s[:, DH:], src_s, dst_s, val_s,
                     sup_t[:, :DH], sup_t[:, DH:], src_t, dst_t, val_t, zeros)
    return (jnp.concatenate([o[0], o[1]], axis=1),
            jnp.concatenate([o[2], o[3]], axis=1))


BM = 400
GRID = N // BM


def _mm(x, w):
    return jnp.dot(x, w, preferred_element_type=jnp.float32)


def _tc_call(body, n_out, *args):
    outs = tuple(jax.ShapeDtypeStruct((N, D), jnp.float32) for _ in range(n_out))
    in_specs = []
    for a in args:
        if a.shape[0] == N:
            in_specs.append(pl.BlockSpec((BM, a.shape[1]), lambda i: (i, 0)))
        else:
            in_specs.append(pl.BlockSpec(a.shape, lambda i: (0,) * a.ndim))
    out_specs = tuple(pl.BlockSpec((BM, D), lambda i: (i, 0)) for _ in range(n_out))
    return pl.pallas_call(
        body,
        grid=(GRID,),
        in_specs=in_specs,
        out_specs=out_specs if n_out > 1 else out_specs[0],
        out_shape=outs if n_out > 1 else outs[0],
    )(*args)


def _t1_body(xs, xt, w0, w1, os_, ot_):
    os_[...] = _mm(xs[...], w0[...])
    ot_[...] = _mm(xt[...], w1[...])


def _t2_body(accs, acct, b0, b1, w2, w3, os_, ot_):
    os_[...] = _mm(_leaky(accs[...] + b0[...]), w2[...])
    ot_[...] = _mm(_leaky(acct[...] + b1[...]), w3[...])


def _t3_body(accs, acct, b2, b3, xs, xt, u0a, u0b, ub0, u1a, u1b, ub1,
             w10, w11, oh, osup, otsup):
    s_ho2 = _leaky(accs[...] + b2[...])
    t_ho2 = _leaky(acct[...] + b3[...])
    sU = _mm(s_ho2, u0a[...]) + _mm(xs[...], u0b[...]) + ub0[...]
    tU = _mm(t_ho2, u1a[...]) + _mm(xt[...], u1b[...]) + ub1[...]
    h = RATE * jnp.maximum(sU, 0.0) + (1.0 - RATE) * jnp.maximum(tU, 0.0)
    oh[...] = h
    osup[...] = _mm(h, w10[...])
    otsup[...] = _mm(h, w11[...])


def _t4_body(accs, acct, b0, b1, os_, ot_):
    os_[...] = _leaky(accs[...] + b0[...])
    ot_[...] = _leaky(acct[...] + b1[...])


def _t5_body(qs, qt, h, w12, b12, w13, b13, w14, b14, w15, b15,
             um0a, um0b, um0c, ul0a, ul0b, ul0c,
             um1a, um1b, um1c, ul1a, ul1b, ul1c,
             omean, osigma):
    hv = h[...]
    s_mean = _leaky(_mm(qs[...], w12[...]) + b12[...])
    s_logstd = _leaky(_mm(qs[...], w13[...]) + b13[...])
    t_mean = _leaky(_mm(qt[...], w14[...]) + b14[...])
    t_logstd = _leaky(_mm(qt[...], w15[...]) + b15[...])
    sUm = _mm(s_mean, um0a[...]) + _mm(hv, um0b[...]) + um0c[...]
    sUl = _mm(s_logstd, ul0a[...]) + _mm(hv, ul0b[...]) + ul0c[...]
    tUm = _mm(t_mean, um1a[...]) + _mm(hv, um1b[...]) + um1c[...]
    tUl = _mm(t_logstd, ul1a[...]) + _mm(hv, ul1b[...]) + ul1c[...]
    omean[...] = RATE * sUm + (1.0 - RATE) * tUm
    osigma[...] = RATE * sUl + (1.0 - RATE) * tUl


def kernel(source_ufea, target_ufea, source_edge_index, target_edge_index,
           source_uv_val, source_vu_val, target_uv_val, target_vu_val,
           L0_gcn_W, L0_gcn_b, L0_union_W, L0_union_b,
           L1_gcn_W, L1_gcn_b, L1_union_W, L1_union_b):
    su = source_edge_index[0].reshape(NS, NCHUNK, CH)
    si = source_edge_index[1].reshape(NS, NCHUNK, CH)
    tu = target_edge_index[0].reshape(NS, NCHUNK, CH)
    ti = target_edge_index[1].reshape(NS, NCHUNK, CH)
    _v16 = lambda v: jnp.broadcast_to(
        v.reshape(NS, NCHUNK, CH, 1), (NS, NCHUNK, CH, 16))
    s_uv = _v16(source_uv_val)
    s_vu = _v16(source_vu_val)
    t_uv = _v16(target_uv_val)
    t_vu = _v16(target_vu_val)
    zeros = jnp.zeros((N, DH), jnp.float32)

    b2 = lambda b: b.reshape(1, D)

    sup_s, sup_t = _tc_call(_t1_body, 2, source_ufea, target_ufea,
                            L0_gcn_W[0], L0_gcn_W[1])
    acc_s, acc_t = _spmm_pair(sup_s, su, si, s_vu, sup_t, tu, ti, t_vu, zeros)
    sup_s, sup_t = _tc_call(_t2_body, 2, acc_s, acc_t,
                            b2(L0_gcn_b[0]), b2(L0_gcn_b[1]),
                            L0_gcn_W[2], L0_gcn_W[3])
    acc_s, acc_t = _spmm_pair(sup_s, si, su, s_uv, sup_t, ti, tu, t_uv, zeros)
    h, sup_s, sup_t = _tc_call(
        _t3_body, 3, acc_s, acc_t, b2(L0_gcn_b[2]), b2(L0_gcn_b[3]),
        source_ufea, target_ufea,
        L0_union_W[0, :D], L0_union_W[0, D:], b2(L0_union_b[0]),
        L0_union_W[1, :D], L0_union_W[1, D:], b2(L0_union_b[1]),
        L1_gcn_W[0], L1_gcn_W[1])

    acc_s, acc_t = _spmm_pair(sup_s, su, si, s_vu, sup_t, tu, ti, t_vu, zeros)
    s_ho, t_ho = _tc_call(_t4_body, 2, acc_s, acc_t,
                          b2(L1_gcn_b[0]), b2(L1_gcn_b[1]))
    q_s, q_t = _spmm_pair(s_ho, si, su, s_uv, t_ho, ti, tu, t_uv, zeros)
    mean, sigma = _tc_call(
        _t5_body, 2, q_s, q_t, h,
        L1_gcn_W[2], b2(L1_gcn_b[2]), L1_gcn_W[3], b2(L1_gcn_b[3]),
        L1_gcn_W[4], b2(L1_gcn_b[4]), L1_gcn_W[5], b2(L1_gcn_b[5]),
        L1_union_W[0, :D], L1_union_W[0, D:], b2(L1_union_b[0]),
        L1_union_W[1, :D], L1_union_W[1, D:], b2(L1_union_b[1]),
        L1_union_W[2, :D], L1_union_W[2, D:], b2(L1_union_b[2]),
        L1_union_W[3, :D], L1_union_W[3, D:], b2(L1_union_b[3]))
    return mean, sigma


# speedup vs baseline: 1.0378x; 1.0378x over previous
import functools

import jax
import jax.numpy as jnp
from jax import lax
from jax.experimental import pallas as pl
from jax.experimental.pallas import tpu as pltpu
from jax.experimental.pallas import tpu_sc as plsc

N = 10000
E = 320000
D = 128
RATE = 0.5
ALPHA = 0.1

NS = 16
CH = 100
EPT = E // NS
NCHUNK = EPT // CH
RPT = 624
TAIL = N - NS * RPT

_MESH = plsc.VectorSubcoreMesh(
    core_axis_name="c", subcore_axis_name="s", num_cores=2, num_subcores=16)


def _leaky(x):
    return jnp.where(x > 0, x, ALPHA * x)


DH = D // 2


NBUF = 5
PREF = 3
NBLK = NCHUNK // NBUF


def _spmm_domain(sup_a, sup_b, src_hbm, dst_hbm, val16_hbm, zeros_hbm,
                 out_a, out_b, idx_src, idx_dst, bufs, vbufs, gs, ss, accum):
    sid = lax.axis_index("s")
    pltpu.sync_copy(src_hbm.at[sid], idx_src)
    pltpu.sync_copy(dst_hbm.at[sid], idx_dst)

    for sup_hbm, out_hbm in ((sup_a, out_a), (sup_b, out_b)):

        def gstart(j, b):
            pltpu.make_async_copy(sup_hbm.at[idx_src.at[j]], bufs[b],
                                  gs[b]).start()
            pltpu.make_async_copy(val16_hbm.at[sid, j], vbufs[b],
                                  gs[b]).start()

        def gwait(j, b):
            pltpu.make_async_copy(sup_hbm.at[idx_src.at[j]], bufs[b],
                                  gs[b]).wait()
            pltpu.make_async_copy(val16_hbm.at[sid, j], vbufs[b],
                                  gs[b]).wait()

        def sstart(j, b):
            pltpu.make_async_copy(bufs[b], accum.at[idx_dst.at[j]],
                                  ss[b]).start(add=True)

        def swait(j, b):
            pltpu.make_async_copy(bufs[b], accum.at[idx_dst.at[j]],
                                  ss[b]).wait()

        def mul(j, b):
            rows = bufs[b]
            vb = vbufs[b]

            def ebody(i, _):
                for u in range(2):
                    e = i * 2 + u
                    v16 = vb[e, :]
                    for k in range(DH // 16):
                        sl = pl.ds(k * 16, 16)
                        rows[e, sl] = rows[e, sl] * v16
                return 0

            lax.fori_loop(0, CH // 2, ebody, 0)

        pltpu.sync_copy(zeros_hbm.at[pl.ds(sid * RPT, RPT)],
                        accum.at[pl.ds(sid * RPT, RPT)])

        @pl.when(sid == NS - 1)
        def _():
            pltpu.sync_copy(zeros_hbm.at[pl.ds(NS * RPT, TAIL)],
                            accum.at[pl.ds(NS * RPT, TAIL)])

        plsc.subcore_barrier()

        for b in range(PREF):
            gstart(b, b)

        def block(g, _):
            j0 = g * NBUF
            for b in range(NBUF):
                j = j0 + b
                gwait(j, b)
                mul(j, b)
                sstart(j, b)
                bn = (b + PREF) % NBUF
                jn = j + PREF

                @pl.when(jn < NCHUNK)
                def _():
                    @pl.when(j >= NBUF - PREF)
                    def _():
                        swait(j - (NBUF - PREF), bn)

                    gstart(jn, bn)
            return 0

        lax.fori_loop(0, NBLK, block, 0)

        for b in range(NBUF):
            swait(NCHUNK - NBUF + b, b)

        plsc.subcore_barrier()
        pltpu.sync_copy(accum.at[pl.ds(sid * RPT, RPT)],
                        out_hbm.at[pl.ds(sid * RPT, RPT)])

        @pl.when(sid == NS - 1)
        def _():
            pltpu.sync_copy(accum.at[pl.ds(NS * RPT, TAIL)],
                            out_hbm.at[pl.ds(NS * RPT, TAIL)])

        plsc.subcore_barrier()


@functools.partial(
    pl.kernel,
    out_type=tuple(jax.ShapeDtypeStruct((N, DH), jnp.float32)
                   for _ in range(4)),
    mesh=_MESH,
    scratch_types=[
        pltpu.VMEM((NCHUNK, CH), jnp.int32),
        pltpu.VMEM((NCHUNK, CH), jnp.int32),
        [pltpu.VMEM((CH, DH), jnp.float32)] * NBUF,
        [pltpu.VMEM((CH, 16), jnp.float32)] * NBUF,
        [pltpu.SemaphoreType.DMA] * NBUF,
        [pltpu.SemaphoreType.DMA] * NBUF,
        pltpu.VMEM_SHARED((N, DH), jnp.float32),
    ],
    compiler_params=pltpu.CompilerParams(use_tc_tiling_on_sc=False),
)
def _spmm_kernel(sup_s_a, sup_s_b, src_s, dst_s, val_s,
                 sup_t_a, sup_t_b, src_t, dst_t, val_t,
                 zeros_hbm, out_s_a, out_s_b, out_t_a, out_t_b,
                 idx_src, idx_dst, bufs, vbufs, gs, ss, accum):
    c = lax.axis_index("c")

    @pl.when(c == 0)
    def _():
        _spmm_domain(sup_s_a, sup_s_b, src_s, dst_s, val_s, zeros_hbm,
                     out_s_a, out_s_b, idx_src, idx_dst, bufs, vbufs, gs, ss,
                     accum)

    @pl.when(c == 1)
    def _():
        _spmm_domain(sup_t_a, sup_t_b, src_t, dst_t, val_t, zeros_hbm,
                     out_t_a, out_t_b, idx_src, idx_dst, bufs, vbufs, gs, ss,
                     accum)


def _spmm_pair(sup_s, src_s, dst_s, val_s, sup_t, src_t, dst_t, val_t, zeros):
    o = _spmm_kernel(sup_s[:, :DH], sup_s[:, DH:], src_s, dst_s, val_s,
                     sup_t[:, :DH], sup_t[:, DH:], src_t, dst_t, val_t, zeros)
    return (jnp.concatenate([o[0], o[1]], axis=1),
            jnp.concatenate([o[2], o[3]], axis=1))


BM = 400
GRID = N // BM


def _mm(x, w):
    return jnp.dot(x, w, preferred_element_type=jnp.float32)


def _tc_call(body, n_out, *args):
    outs = tuple(jax.ShapeDtypeStruct((N, D), jnp.float32) for _ in range(n_out))
    in_specs = []
    for a in args:
        if a.shape[0] == N:
            in_specs.append(pl.BlockSpec((BM, a.shape[1]), lambda i: (i, 0)))
        else:
            in_specs.append(pl.BlockSpec(a.shape, lambda i: (0,) * a.ndim))
    out_specs = tuple(pl.BlockSpec((BM, D), lambda i: (i, 0)) for _ in range(n_out))
    return pl.pallas_call(
        body,
        grid=(GRID,),
        in_specs=in_specs,
        out_specs=out_specs if n_out > 1 else out_specs[0],
        out_shape=outs if n_out > 1 else outs[0],
    )(*args)


def _t1_body(xs, xt, w0, w1, os_, ot_):
    os_[...] = _mm(xs[...], w0[...])
    ot_[...] = _mm(xt[...], w1[...])


def _t2_body(accs, acct, b0, b1, w2, w3, os_, ot_):
    os_[...] = _mm(_leaky(accs[...] + b0[...]), w2[...])
    ot_[...] = _mm(_leaky(acct[...] + b1[...]), w3[...])


def _t3_body(accs, acct, b2, b3, xs, xt, u0a, u0b, ub0, u1a, u1b, ub1,
             w10, w11, oh, osup, otsup):
    s_ho2 = _leaky(accs[...] + b2[...])
    t_ho2 = _leaky(acct[...] + b3[...])
    sU = _mm(s_ho2, u0a[...]) + _mm(xs[...], u0b[...]) + ub0[...]
    tU = _mm(t_ho2, u1a[...]) + _mm(xt[...], u1b[...]) + ub1[...]
    h = RATE * jnp.maximum(sU, 0.0) + (1.0 - RATE) * jnp.maximum(tU, 0.0)
    oh[...] = h
    osup[...] = _mm(h, w10[...])
    otsup[...] = _mm(h, w11[...])


def _t4_body(accs, acct, b0, b1, os_, ot_):
    os_[...] = _leaky(accs[...] + b0[...])
    ot_[...] = _leaky(acct[...] + b1[...])


def _t5_body(qs, qt, h, w12, b12, w13, b13, w14, b14, w15, b15,
             um0a, um0b, um0c, ul0a, ul0b, ul0c,
             um1a, um1b, um1c, ul1a, ul1b, ul1c,
             omean, osigma):
    hv = h[...]
    s_mean = _leaky(_mm(qs[...], w12[...]) + b12[...])
    s_logstd = _leaky(_mm(qs[...], w13[...]) + b13[...])
    t_mean = _leaky(_mm(qt[...], w14[...]) + b14[...])
    t_logstd = _leaky(_mm(qt[...], w15[...]) + b15[...])
    sUm = _mm(s_mean, um0a[...]) + _mm(hv, um0b[...]) + um0c[...]
    sUl = _mm(s_logstd, ul0a[...]) + _mm(hv, ul0b[...]) + ul0c[...]
    tUm = _mm(t_mean, um1a[...]) + _mm(hv, um1b[...]) + um1c[...]
    tUl = _mm(t_logstd, ul1a[...]) + _mm(hv, ul1b[...]) + ul1c[...]
    omean[...] = RATE * sUm + (1.0 - RATE) * tUm
    osigma[...] = RATE * sUl + (1.0 - RATE) * tUl


def kernel(source_ufea, target_ufea, source_edge_index, target_edge_index,
           source_uv_val, source_vu_val, target_uv_val, target_vu_val,
           L0_gcn_W, L0_gcn_b, L0_union_W, L0_union_b,
           L1_gcn_W, L1_gcn_b, L1_union_W, L1_union_b):
    su = source_edge_index[0].reshape(NS, NCHUNK, CH)
    si = source_edge_index[1].reshape(NS, NCHUNK, CH)
    tu = target_edge_index[0].reshape(NS, NCHUNK, CH)
    ti = target_edge_index[1].reshape(NS, NCHUNK, CH)
    _v16 = lambda v: jnp.broadcast_to(
        v.reshape(NS, NCHUNK, CH, 1), (NS, NCHUNK, CH, 16))
    s_uv = _v16(source_uv_val)
    s_vu = _v16(source_vu_val)
    t_uv = _v16(target_uv_val)
    t_vu = _v16(target_vu_val)
    zeros = jnp.zeros((N, DH), jnp.float32)

    b2 = lambda b: b.reshape(1, D)

    sup_s, sup_t = _tc_call(_t1_body, 2, source_ufea, target_ufea,
                            L0_gcn_W[0], L0_gcn_W[1])
    acc_s, acc_t = _spmm_pair(sup_s, su, si, s_vu, sup_t, tu, ti, t_vu, zeros)
    sup_s, sup_t = _tc_call(_t2_body, 2, acc_s, acc_t,
                            b2(L0_gcn_b[0]), b2(L0_gcn_b[1]),
                            L0_gcn_W[2], L0_gcn_W[3])
    acc_s, acc_t = _spmm_pair(sup_s, si, su, s_uv, sup_t, ti, tu, t_uv, zeros)
    h, sup_s, sup_t = _tc_call(
        _t3_body, 3, acc_s, acc_t, b2(L0_gcn_b[2]), b2(L0_gcn_b[3]),
        source_ufea, target_ufea,
        L0_union_W[0, :D], L0_union_W[0, D:], b2(L0_union_b[0]),
        L0_union_W[1, :D], L0_union_W[1, D:], b2(L0_union_b[1]),
        L1_gcn_W[0], L1_gcn_W[1])

    acc_s, acc_t = _spmm_pair(sup_s, su, si, s_vu, sup_t, tu, ti, t_vu, zeros)
    s_ho, t_ho = _tc_call(_t4_body, 2, acc_s, acc_t,
                          b2(L1_gcn_b[0]), b2(L1_gcn_b[1]))
    q_s, q_t = _spmm_pair(s_ho, si, su, s_uv, t_ho, ti, tu, t_uv, zeros)
    mean, sigma = _tc_call(
        _t5_body, 2, q_s, q_t, h,
        L1_gcn_W[2], b2(L1_gcn_b[2]), L1_gcn_W[3], b2(L1_gcn_b[3]),
        L1_gcn_W[4], b2(L1_gcn_b[4]), L1_gcn_W[5], b2(L1_gcn_b[5]),
        L1_union_W[0, :D], L1_union_W[0, D:], b2(L1_union_b[0]),
        L1_union_W[1, :D], L1_union_W[1, D:], b2(L1_union_b[1]),
        L1_union_W[2, :D], L1_union_W[2, D:], b2(L1_union_b[2]),
        L1_union_W[3, :D], L1_union_W[3, D:], b2(L1_union_b[3]))
    return mean, sigma
